# baseline (speedup 1.0000x reference)
import jax
import jax.numpy as jnp
from jax import lax
from jax.experimental import pallas as pl
from jax.experimental.pallas import tpu as pltpu
from jax.experimental.pallas import tpu_sc as plsc

N_ROWS = 100000
TYPES = 100
DIM = 128
NW = 32
W = 3200
CH = 128
NCH = W // CH
LANES = 16


def _emb_body(idx_hbm, table_hbm, out_hbm, idx_v, table_v, bufs, lsem, wsem0, wsem1):
    wid = lax.axis_index("s") * 2 + lax.axis_index("c")
    base = jnp.minimum(wid * W, N_ROWS - W)

    cp_t = pltpu.async_copy(table_hbm, table_v, lsem)
    cp_i = pltpu.async_copy(idx_hbm.at[pl.ds(base, W)], idx_v, lsem)
    cp_t.wait()
    cp_i.wait()

    iota = lax.iota(jnp.int32, LANES)
    seg_off = [jnp.full((LANES,), 16 * k, jnp.int32) + iota for k in range(DIM // LANES)]

    def fill(c, b):
        for rg in range(CH // LANES):
            idx16 = idx_v[pl.ds(c * CH + rg * LANES, LANES)]
            for r in range(LANES):
                row = jnp.take_along_axis(
                    idx16, jnp.full((LANES,), r, jnp.int32), axis=0,
                    mode="promise_in_bounds")
                dstr = b * CH + rg * LANES + r
                for k in range(DIM // LANES):
                    v = plsc.load_gather(table_v, [row, seg_off[k]])
                    bufs[dstr, pl.ds(16 * k, LANES)] = v

    def write(c, b, sem):
        return pltpu.make_async_copy(
            bufs.at[pl.ds(b * CH, CH)],
            out_hbm.at[pl.ds(base + c * CH, CH)],
            sem,
        )

    def step(c, carry):
        par = lax.rem(c, 2)

        @pl.when(jnp.logical_and(c >= 2, par == 0))
        def _():
            write(c - 2, 0, wsem0).wait()

        @pl.when(jnp.logical_and(c >= 2, par == 1))
        def _():
            write(c - 2, 1, wsem1).wait()

        fill(c, par)

        @pl.when(par == 0)
        def _():
            write(c, 0, wsem0).start()

        @pl.when(par == 1)
        def _():
            write(c, 1, wsem1).start()

        return carry

    lax.fori_loop(0, NCH, step, 0)
    write(NCH - 2, 1, wsem1).wait()
    write(NCH - 1, 0, wsem0).wait()


@jax.jit
def kernel(node_type, table):
    mesh = plsc.VectorSubcoreMesh(core_axis_name="c", subcore_axis_name="s")
    k = pl.kernel(
        _emb_body,
        out_type=jax.ShapeDtypeStruct((N_ROWS, DIM), jnp.float32),
        mesh=mesh,
        compiler_params=pltpu.CompilerParams(needs_layout_passes=False),
        scratch_types=[
            pltpu.VMEM((W,), jnp.int32),
            pltpu.VMEM((TYPES, DIM), jnp.float32),
            pltpu.VMEM((2 * CH, DIM), jnp.float32),
            pltpu.SemaphoreType.DMA,
            pltpu.SemaphoreType.DMA,
            pltpu.SemaphoreType.DMA,
        ],
    )
    return k(node_type.astype(jnp.int32), table)


# speedup vs baseline: 1.6998x; 1.6998x over previous
import jax
import jax.numpy as jnp
from jax import lax
from jax.experimental import pallas as pl
from jax.experimental.pallas import tpu as pltpu
from jax.experimental.pallas import tpu_sc as plsc

N_ROWS = 100000
TYPES = 100
DIM = 128
NW = 32
W = 3200
CH = 128
NCH = W // CH
LANES = 16


def _emb_body(idx_hbm, table_hbm, out_hbm, idx_v, table_v, bufs, lsem, wsem0, wsem1):
    wid = lax.axis_index("s") * 2 + lax.axis_index("c")
    base = jnp.minimum(wid * W, N_ROWS - W)

    cp_t = pltpu.async_copy(table_hbm, table_v, lsem)
    cp_i = pltpu.async_copy(idx_hbm.at[pl.ds(base, W)], idx_v, lsem)
    cp_t.wait()
    cp_i.wait()

    iota = lax.iota(jnp.int32, LANES)
    seg_off = [jnp.full((LANES,), 16 * k, jnp.int32) + iota for k in range(DIM // LANES)]

    def fill(c, b):
        for rg in range(CH // LANES):
            idx16 = idx_v[pl.ds(c * CH + rg * LANES, LANES)]
            rows = [
                jnp.take_along_axis(idx16, jnp.full((LANES,), r, jnp.int32),
                                    axis=0, mode="promise_in_bounds")
                for r in range(LANES)
            ]
            for k in range(DIM // LANES):
                vs = [plsc.load_gather(table_v, [rows[r], seg_off[k]])
                      for r in range(LANES)]
                for r in range(LANES):
                    bufs[b * CH + rg * LANES + r, pl.ds(16 * k, LANES)] = vs[r]

    def write(c, b, sem):
        return pltpu.make_async_copy(
            bufs.at[pl.ds(b * CH, CH)],
            out_hbm.at[pl.ds(base + c * CH, CH)],
            sem,
        )

    def step(c, carry):
        par = lax.rem(c, 2)

        @pl.when(jnp.logical_and(c >= 2, par == 0))
        def _():
            write(c - 2, 0, wsem0).wait()

        @pl.when(jnp.logical_and(c >= 2, par == 1))
        def _():
            write(c - 2, 1, wsem1).wait()

        fill(c, par)

        @pl.when(par == 0)
        def _():
            write(c, 0, wsem0).start()

        @pl.when(par == 1)
        def _():
            write(c, 1, wsem1).start()

        return carry

    lax.fori_loop(0, NCH, step, 0)
    write(NCH - 2, 1, wsem1).wait()
    write(NCH - 1, 0, wsem0).wait()


@jax.jit
def kernel(node_type, table):
    mesh = plsc.VectorSubcoreMesh(core_axis_name="c", subcore_axis_name="s")
    k = pl.kernel(
        _emb_body,
        out_type=jax.ShapeDtypeStruct((N_ROWS, DIM), jnp.float32),
        mesh=mesh,
        compiler_params=pltpu.CompilerParams(needs_layout_passes=False),
        scratch_types=[
            pltpu.VMEM((W,), jnp.int32),
            pltpu.VMEM((TYPES, DIM), jnp.float32),
            pltpu.VMEM((2 * CH, DIM), jnp.float32),
            pltpu.SemaphoreType.DMA,
            pltpu.SemaphoreType.DMA,
            pltpu.SemaphoreType.DMA,
        ],
    )
    return k(node_type.astype(jnp.int32), table)


# speedup vs baseline: 4.3145x; 2.5383x over previous
import jax
import jax.numpy as jnp
from jax import lax
from jax.experimental import pallas as pl
from jax.experimental.pallas import tpu as pltpu
from jax.experimental.pallas import tpu_sc as plsc

N_ROWS = 100000
DIM = 128
NW = 32
W = 3200
CH = 80
NCH = W // CH
NBUF = 8
STEPS = NCH // NBUF


def _emb_body(idx_hbm, table_hbm, out_hbm, idx_v, table_sh, bufs, gsem, wsem, lsem):
    wid = lax.axis_index("s") * 2 + lax.axis_index("c")
    base = jnp.minimum(wid * W, N_ROWS - W)

    cp_i = pltpu.async_copy(idx_hbm.at[pl.ds(base, W)], idx_v, lsem)

    @pl.when(lax.axis_index("s") == 0)
    def _():
        pltpu.sync_copy(table_hbm, table_sh)

    cp_i.wait()
    plsc.subcore_barrier()

    def gather(c, b):
        return pltpu.make_async_copy(
            table_sh.at[idx_v.at[pl.ds(c * CH, CH)]],
            bufs.at[pl.ds(b * CH, CH)],
            gsem.at[b],
        )

    def write(c, b):
        return pltpu.make_async_copy(
            bufs.at[pl.ds(b * CH, CH)],
            out_hbm.at[pl.ds(base + c * CH, CH)],
            wsem.at[b],
        )

    for b in range(NBUF):
        gather(b, b).start()

    def step(s, carry):
        for b in range(NBUF):
            c = s * NBUF + b
            gather(c, b).wait()
            write(c, b).start()
        for b in range(NBUF):
            c = s * NBUF + b
            write(c, b).wait()

            @pl.when(c + NBUF < NCH)
            def _():
                gather(c + NBUF, b).start()

        return carry

    lax.fori_loop(0, STEPS, step, 0)


@jax.jit
def kernel(node_type, table):
    mesh = plsc.VectorSubcoreMesh(core_axis_name="c", subcore_axis_name="s")
    k = pl.kernel(
        _emb_body,
        out_type=jax.ShapeDtypeStruct((N_ROWS, DIM), jnp.float32),
        mesh=mesh,
        scratch_types=[
            pltpu.VMEM((W,), jnp.int32),
            pltpu.VMEM_SHARED((100, DIM), jnp.float32),
            pltpu.VMEM((NBUF * CH, DIM), jnp.float32),
            pltpu.SemaphoreType.DMA((NBUF,)),
            pltpu.SemaphoreType.DMA((NBUF,)),
            pltpu.SemaphoreType.DMA,
        ],
    )
    return k(node_type.astype(jnp.int32), table)
